# baseline (speedup 1.0000x reference)
import functools

import jax
import jax.numpy as jnp
from jax import lax
from jax.experimental import pallas as pl
from jax.experimental.pallas import tpu as pltpu
from jax.experimental.pallas import tpu_sc as plsc

N = 10000
E = 160000
D = 256
NT_N = 10
TN = N // NT_N
NT_E = 160
TE = E // NT_E

_NEG_INF = float("-inf")


def _enc_body(nodes_ref, coords_ref, w1n, b1n, w2n, b2n, w1c, b1c, w2c, b2c,
              h_ref, hc_ref):
    t = jnp.maximum(
        jnp.dot(nodes_ref[...], w1n[...], preferred_element_type=jnp.float32)
        + b1n[...], 0.0)
    h_ref[...] = jnp.dot(t, w2n[...], preferred_element_type=jnp.float32) + b2n[...]
    t2 = jnp.maximum(
        jnp.dot(coords_ref[...], w1c[...], preferred_element_type=jnp.float32)
        + b1c[...], 0.0)
    hc_ref[...] = jnp.dot(t2, w2c[...], preferred_element_type=jnp.float32) + b2c[...]


def _encode(nodes, coords8, hn_W1, hn_b1, hn_W2, hn_b2, hc_W1p, hc_b1, hc_W2, hc_b2):
    full = lambda shape: pl.BlockSpec(shape, lambda i: (0, 0))
    return pl.pallas_call(
        _enc_body,
        grid=(NT_N,),
        in_specs=[
            pl.BlockSpec((TN, 128), lambda i: (i, 0)),
            pl.BlockSpec((TN, 8), lambda i: (i, 0)),
            full((128, D)), full((1, D)), full((D, D)), full((1, D)),
            full((8, D)), full((1, D)), full((D, D)), full((1, D)),
        ],
        out_specs=[
            pl.BlockSpec((TN, D), lambda i: (i, 0)),
            pl.BlockSpec((TN, D), lambda i: (i, 0)),
        ],
        out_shape=[
            jax.ShapeDtypeStruct((N, D), jnp.float32),
            jax.ShapeDtypeStruct((N, D), jnp.float32),
        ],
    )(nodes, coords8, hn_W1, hn_b1.reshape(1, D), hn_W2, hn_b2.reshape(1, D),
      hc_W1p, hc_b1.reshape(1, D), hc_W2, hc_b2.reshape(1, D))


def _st_body(h_ref, hc_ref, wa, wb, wc, b1, s_ref, t_ref):
    h = h_ref[...]
    hcwc = jnp.dot(hc_ref[...], wc[...], preferred_element_type=jnp.float32)
    s_ref[...] = (jnp.dot(h, wa[...], preferred_element_type=jnp.float32)
                  + hcwc + b1[...]).astype(jnp.bfloat16)
    t_ref[...] = (jnp.dot(h, wb[...], preferred_element_type=jnp.float32)
                  - hcwc).astype(jnp.bfloat16)


def _node_transform(h, hc, wa, wb, wc, b1):
    full = lambda: pl.BlockSpec((D, D), lambda i: (0, 0))
    return pl.pallas_call(
        _st_body,
        grid=(NT_N,),
        in_specs=[
            pl.BlockSpec((TN, D), lambda i: (i, 0)),
            pl.BlockSpec((TN, D), lambda i: (i, 0)),
            full(), full(), full(),
            pl.BlockSpec((1, D), lambda i: (0, 0)),
        ],
        out_specs=[
            pl.BlockSpec((TN, D), lambda i: (i, 0)),
            pl.BlockSpec((TN, D), lambda i: (i, 0)),
        ],
        out_shape=[
            jax.ShapeDtypeStruct((N, D), jnp.bfloat16),
            jax.ShapeDtypeStruct((N, D), jnp.bfloat16),
        ],
    )(h, hc, wa, wb, wc, b1.reshape(1, D))


_CH = 200


def _gather2(Sv, Tv, src, dst):
    info = plsc.get_sparse_core_info()
    nc, ns = info.num_cores, info.num_subcores
    nw = nc * ns
    epw = E // nw
    nch = epw // _CH
    hd = D // 2
    mesh = plsc.VectorSubcoreMesh(core_axis_name="c", subcore_axis_name="s")

    @functools.partial(
        pl.kernel,
        out_type=(jax.ShapeDtypeStruct((E, hd), jnp.float32),
                  jax.ShapeDtypeStruct((E, hd), jnp.float32)),
        mesh=mesh,
        scratch_types=[
            pltpu.VMEM((_CH,), jnp.int32), pltpu.VMEM((_CH,), jnp.int32),
            pltpu.VMEM((_CH,), jnp.int32), pltpu.VMEM((_CH,), jnp.int32),
            pltpu.VMEM((_CH, hd), jnp.float32), pltpu.VMEM((_CH, hd), jnp.float32),
            pltpu.VMEM((_CH, hd), jnp.float32), pltpu.VMEM((_CH, hd), jnp.float32),
            pltpu.SemaphoreType.DMA, pltpu.SemaphoreType.DMA,
            pltpu.SemaphoreType.DMA, pltpu.SemaphoreType.DMA,
            pltpu.SemaphoreType.DMA, pltpu.SemaphoreType.DMA,
        ],
    )
    def k(S_hbm, T_hbm, src_hbm, dst_hbm, P_hbm, Q_hbm,
          si0, si1, di0, di1, sr0, sr1, dr0, dr1,
          semi0, semi1, semg0, semg1, semo0, semo1):
        wid = lax.axis_index("s") * nc + lax.axis_index("c")
        base_w = wid * epw
        si = (si0, si1); di = (di0, di1)
        sr = (sr0, sr1); dr = (dr0, dr1)
        semi = (semi0, semi1); semg = (semg0, semg1); semo = (semo0, semo1)

        def start_idx(c, sl):
            base = base_w + c * _CH
            return (pltpu.async_copy(src_hbm.at[pl.ds(base, _CH)], si[sl], semi[sl]),
                    pltpu.async_copy(dst_hbm.at[pl.ds(base, _CH)], di[sl], semi[sl]))

        def start_gather(sl):
            return (pltpu.async_copy(S_hbm.at[si[sl]], sr[sl], semg[sl]),
                    pltpu.async_copy(T_hbm.at[di[sl]], dr[sl], semg[sl]))

        def start_out(c, sl):
            base = base_w + c * _CH
            return (pltpu.async_copy(sr[sl], P_hbm.at[pl.ds(base, _CH)], semo[sl]),
                    pltpu.async_copy(dr[sl], Q_hbm.at[pl.ds(base, _CH)], semo[sl]))

        g = {}; o = {}; idx = {}
        idx[0] = start_idx(0, 0)
        for cp in idx[0]:
            cp.wait()
        g[0] = start_gather(0)
        if nch > 1:
            idx[1] = start_idx(1, 1)
        for c in range(nch):
            sl = c & 1
            if c + 1 < nch:
                for cp in idx[c + 1]:
                    cp.wait()
                if c - 1 >= 0:
                    for cp in o[c - 1]:
                        cp.wait()
                g[c + 1] = start_gather(1 - sl)
            for cp in g[c]:
                cp.wait()
            o[c] = start_out(c, sl)
            if c + 2 < nch:
                idx[c + 2] = start_idx(c + 2, sl)
        for cc in (nch - 2, nch - 1):
            if cc >= 0 and cc in o:
                for cp in o[cc]:
                    cp.wait()

    return k(Sv, Tv, src, dst)


def _edge_body(p_ref, q_ref, w2, b2, m_ref):
    a = jnp.maximum(p_ref[...].astype(jnp.float32) + q_ref[...].astype(jnp.float32),
                    0.0).astype(jnp.bfloat16)
    m_ref[...] = (jnp.dot(a, w2[...], preferred_element_type=jnp.float32)
                  + b2[...]).astype(jnp.bfloat16)


def _edge_mlp(P, Q, w2, b2):
    return pl.pallas_call(
        _edge_body,
        grid=(NT_E,),
        in_specs=[
            pl.BlockSpec((TE, D), lambda i: (i, 0)),
            pl.BlockSpec((TE, D), lambda i: (i, 0)),
            pl.BlockSpec((D, D), lambda i: (0, 0)),
            pl.BlockSpec((1, D), lambda i: (0, 0)),
        ],
        out_specs=pl.BlockSpec((TE, D), lambda i: (i, 0)),
        out_shape=jax.ShapeDtypeStruct((E, D), jnp.bfloat16),
    )(P, Q, w2, b2.reshape(1, D))


_WR = 320
_SCH = 2000
_CB = 64
_RING = 4096
_FL = 2048
_PLN = E + 2 * _FL
_PAD_PK = _WR


def _sc_mesh_info():
    info = plsc.get_sparse_core_info()
    return info.num_cores, info.num_subcores


def _scatter_plan(dst):
    nc, ns = _sc_mesh_info()
    nw = nc * ns
    nchunks = E // _SCH
    mesh = plsc.VectorSubcoreMesh(core_axis_name="c", subcore_axis_name="s")

    @functools.partial(
        pl.kernel,
        out_type=(jax.ShapeDtypeStruct((nw * _PLN,), jnp.int32),
                  jax.ShapeDtypeStruct((nw * 8,), jnp.int32)),
        mesh=mesh,
        compiler_params=pltpu.CompilerParams(needs_layout_passes=False),
        scratch_types=[
            pltpu.VMEM((_RING,), jnp.int32),
            pltpu.VMEM((_SCH,), jnp.int32),
            pltpu.VMEM((16,), jnp.int32),
        ],
    )
    def k(dst_hbm, plan_hbm, cnt_hbm, ring, dstv, cbuf):
        wid = lax.axis_index("s") * nc + lax.axis_index("c")
        lo = wid * _WR
        base_out = wid * _PLN
        iota = lax.iota(jnp.int32, 16)

        def flush(flushed):
            fl8 = pl.multiple_of(flushed, _FL)
            half = (fl8 >> 11) & 1
            pltpu.sync_copy(ring.at[pl.ds(half * _FL, _FL)],
                            plan_hbm.at[pl.ds(base_out + fl8, _FL)])
            return flushed + _FL

        def chunk(c, carry):
            cntv, flushed = carry
            pltpu.sync_copy(dst_hbm.at[pl.ds(c * _SCH, _SCH)], dstv)

            def scan_g(g, cntv):
                ld = dstv[pl.ds(g * 16, 16)] - lo
                msk = (ld >= 0) & (ld < _WR)
                pk = ((c * _SCH + g * 16 + iota) << 9) | (ld & 511)
                pos = (cntv + plsc.cumsum(msk.astype(jnp.int32)) - 1) & (_RING - 1)
                plsc.store_scatter(ring, [pos], pk, mask=msk)
                return cntv + plsc.all_reduce_population_count(msk)

            cntv = lax.fori_loop(0, _SCH // 16, scan_g, cntv)
            cnt_s = lax.reduce_max(cntv, (0,))

            def maybe_flush(flushed):
                return lax.cond(cnt_s - flushed >= _FL, flush,
                                lambda f: f, flushed)

            return cntv, maybe_flush(flushed)

        cntv, flushed = lax.fori_loop(
            0, nchunks, chunk,
            (jnp.zeros((16,), jnp.int32), jnp.int32(0)))
        cnt_s = lax.reduce_max(cntv, (0,))
        padv = jnp.full((16,), _PAD_PK, jnp.int32)
        for kk in range(4):
            plsc.store_scatter(ring, [(cnt_s + kk * 16 + iota) & (_RING - 1)], padv)

        def cond(fl):
            return fl < cnt_s + _CB

        flushed = lax.while_loop(cond, flush, flushed)
        cbuf[pl.ds(0, 16)] = cntv
        pltpu.sync_copy(cbuf.at[pl.ds(0, 8)], cnt_hbm.at[pl.ds(wid * 8, 8)])

    return k(dst)


def _scatter_max2(Mv, plan, counts):
    nc, ns = _sc_mesh_info()
    nw = nc * ns
    hd = D // 2
    mesh = plsc.VectorSubcoreMesh(core_axis_name="c", subcore_axis_name="s")

    @functools.partial(
        pl.kernel,
        out_type=jax.ShapeDtypeStruct((nw * _WR * D,), jnp.float32),
        mesh=mesh,
        compiler_params=pltpu.CompilerParams(needs_layout_passes=False),
        scratch_types=(
            [pltpu.VMEM(((_WR + 1) * 16,), jnp.float32) for _ in range(16)]
            + [
                pltpu.VMEM((_CB,), jnp.int32),
                pltpu.VMEM((_CB,), jnp.int32),
                pltpu.VMEM((_CB,), jnp.int32),
                pltpu.VMEM((_CB,), jnp.int32),
                pltpu.VMEM((_CB,), jnp.int32),
                pltpu.VMEM((_CB,), jnp.int32),
                pltpu.VMEM((_CB, hd), jnp.float32),
                pltpu.VMEM((_CB, hd), jnp.float32),
                pltpu.VMEM((16,), jnp.int32),
                pltpu.VMEM((40 * D,), jnp.float32),
                pltpu.SemaphoreType.DMA,
                pltpu.SemaphoreType.DMA,
            ]
        ),
    )
    def k(M_hbm, plan_hbm, cnt_hbm, agg_hbm, *rest):
        accs = rest[:16]
        (pke0, pke1, mid0, mid1, mld0, mld1, rows0, rows1, cbuf, stg,
         sem0, sem1) = rest[16:]
        wid = lax.axis_index("s") * nc + lax.axis_index("c")
        lo = wid * _WR
        base_in = wid * _PLN
        iota = lax.iota(jnp.int32, 16)
        neg = jnp.full((16,), _NEG_INF, jnp.float32)
        m16 = jnp.full((16,), -65536, jnp.int32)

        def initacc(i, _):
            for b in range(16):
                plsc.store_scatter(accs[b], [jnp.full((16,), i * 16, jnp.int32) + iota], neg)
            return 0
        lax.fori_loop(0, _WR + 1, initacc, 0)

        cbuf[pl.ds(0, 16)] = jnp.zeros((16,), jnp.int32)
        pltpu.sync_copy(cnt_hbm.at[pl.ds(wid * 8, 8)], cbuf.at[pl.ds(0, 8)])
        cnt = lax.reduce_max(cbuf[pl.ds(0, 16)], (0,))
        nbat = (cnt + _CB - 1) // _CB

        slots = ((pke0, mid0, mld0, rows0, sem0),
                 (pke1, mid1, mld1, rows1, sem1))

        def stage(b, slot):
            pke, mid, mld, rows, sem = slots[slot]
            pltpu.sync_copy(plan_hbm.at[pl.ds(base_in + b * _CB, _CB)], pke)
            for i in range(_CB // 16):
                pk = pke[pl.ds(i * 16, 16)]
                mid[pl.ds(i * 16, 16)] = pk >> 9
                mld[pl.ds(i * 16, 16)] = pk & 511
            return pltpu.async_copy(M_hbm.at[mid], rows, sem)

        def process(slot):
            _, _, mld, rows, _ = slots[slot]

            def row(r, _):
                rfull = jnp.full((16,), r, jnp.int32)
                lds = plsc.load_gather(mld, [rfull])
                lds = jnp.minimum(lds, _WR)
                for j in range(hd // 16):
                    w = plsc.load_gather(rows, [rfull, iota + j * 16])
                    wi = plsc.bitcast(w, jnp.int32)
                    lov = plsc.bitcast(wi << 16, jnp.float32)
                    hiv = plsc.bitcast(wi & m16, jnp.float32)
                    addr = lds * 16 + iota
                    for par, val in ((0, lov), (1, hiv)):
                        ref = accs[2 * j + par]
                        a = plsc.load_gather(ref, [addr])
                        plsc.store_scatter(ref, [addr], jnp.maximum(a, val))
                return 0

            lax.fori_loop(0, _CB, row, 0)

        @pl.when(nbat > 0)
        def _run():
            stage(0, 0)

            def pair(i, _):
                b1 = 2 * i + 1

                @pl.when(b1 < nbat)
                def _s1():
                    stage(b1, 1)

                pltpu.make_async_copy(M_hbm.at[slots[0][1]], slots[0][3],
                                      slots[0][4]).wait()
                process(0)

                @pl.when(b1 + 1 < nbat)
                def _s0():
                    stage(b1 + 1, 0)

                @pl.when(b1 < nbat)
                def _p1():
                    pltpu.make_async_copy(M_hbm.at[slots[1][1]], slots[1][3],
                                          slots[1][4]).wait()
                    process(1)
                return 0

            lax.fori_loop(0, (nbat + 1) // 2, pair, 0)

        def rowchunk(rc, _):
            def rowcp(r2, _):
                rowv = jnp.full((16,), (rc * 40 + r2) * 16, jnp.int32) + iota
                for j in range(8):
                    for par in range(2):
                        v = plsc.load_gather(accs[2 * j + par], [rowv])
                        plsc.store_scatter(
                            stg, [r2 * D + 32 * j + 2 * iota + par], v)
                return 0

            lax.fori_loop(0, 40, rowcp, 0)
            pltpu.sync_copy(
                stg, agg_hbm.at[pl.ds((lo + rc * 40) * D, 40 * D)])
            return 0

        lax.fori_loop(0, _WR // 40, rowchunk, 0)

    return k(Mv, plan, counts)


def _upd_body(h_ref, agg_ref, out_ref):
    agg = agg_ref[...]
    out_ref[...] = h_ref[...] + jnp.where(jnp.isneginf(agg), 0.0, agg)


def _h_update(h, agg):
    return pl.pallas_call(
        _upd_body,
        grid=(NT_N,),
        in_specs=[
            pl.BlockSpec((TN, D), lambda i: (i, 0)),
            pl.BlockSpec((TN, D), lambda i: (i, 0)),
        ],
        out_specs=pl.BlockSpec((TN, D), lambda i: (i, 0)),
        out_shape=jax.ShapeDtypeStruct((N, D), jnp.float32),
    )(h, agg)


def kernel(nodes, coords, edge_index, hn_W1, hn_b1, hn_W2, hn_b2,
           hc_W1, hc_b1, hc_W2, hc_b2, mp_W1, mp_b1, mp_W2, mp_b2):
    src = edge_index[0]
    dst = edge_index[1]
    coords8 = jnp.pad(coords, ((0, 0), (0, 5)))
    hc_W1p = jnp.pad(hc_W1, ((0, 5), (0, 0)))

    h, hcv = _encode(nodes, coords8, hn_W1, hn_b1, hn_W2, hn_b2,
                     hc_W1p, hc_b1, hc_W2, hc_b2)

    plan, counts = _scatter_plan(dst)

    for i in range(3):
        wa = mp_W1[i, 0:D, :]
        wb = mp_W1[i, D:2 * D, :]
        wc = mp_W1[i, 2 * D:3 * D, :]
        S, T = _node_transform(h, hcv, wa, wb, wc, mp_b1[i])
        Sv = lax.bitcast_convert_type(S.reshape(N, D // 2, 2), jnp.float32)
        Tv = lax.bitcast_convert_type(T.reshape(N, D // 2, 2), jnp.float32)
        P, Q = _gather2(Sv, Tv, src, dst)
        P16 = lax.bitcast_convert_type(P, jnp.bfloat16).reshape(E, D)
        Q16 = lax.bitcast_convert_type(Q, jnp.bfloat16).reshape(E, D)
        M = _edge_mlp(P16, Q16, mp_W2[i].astype(jnp.bfloat16), mp_b2[i])
        Mv = lax.bitcast_convert_type(M.reshape(E, D // 2, 2), jnp.float32)
        agg = _scatter_max2(Mv, plan, counts).reshape(-1, D)[:N]
        h = _h_update(h, agg)
    return h


# speedup vs baseline: 3.4953x; 3.4953x over previous
import functools

import jax
import jax.numpy as jnp
from jax import lax
from jax.experimental import pallas as pl
from jax.experimental.pallas import tpu as pltpu
from jax.experimental.pallas import tpu_sc as plsc

N = 10000
E = 160000
D = 256
NT_N = 10
TN = N // NT_N
NT_E = 160
TE = E // NT_E

_NEG_INF = float("-inf")


def _enc_body(nodes_ref, coords_ref, w1n, b1n, w2n, b2n, w1c, b1c, w2c, b2c,
              h_ref, hc_ref):
    t = jnp.maximum(
        jnp.dot(nodes_ref[...], w1n[...], preferred_element_type=jnp.float32)
        + b1n[...], 0.0)
    h_ref[...] = jnp.dot(t, w2n[...], preferred_element_type=jnp.float32) + b2n[...]
    t2 = jnp.maximum(
        jnp.dot(coords_ref[...], w1c[...], preferred_element_type=jnp.float32)
        + b1c[...], 0.0)
    hc_ref[...] = jnp.dot(t2, w2c[...], preferred_element_type=jnp.float32) + b2c[...]


def _encode(nodes, coords8, hn_W1, hn_b1, hn_W2, hn_b2, hc_W1p, hc_b1, hc_W2, hc_b2):
    full = lambda shape: pl.BlockSpec(shape, lambda i: (0, 0))
    return pl.pallas_call(
        _enc_body,
        grid=(NT_N,),
        in_specs=[
            pl.BlockSpec((TN, 128), lambda i: (i, 0)),
            pl.BlockSpec((TN, 8), lambda i: (i, 0)),
            full((128, D)), full((1, D)), full((D, D)), full((1, D)),
            full((8, D)), full((1, D)), full((D, D)), full((1, D)),
        ],
        out_specs=[
            pl.BlockSpec((TN, D), lambda i: (i, 0)),
            pl.BlockSpec((TN, D), lambda i: (i, 0)),
        ],
        out_shape=[
            jax.ShapeDtypeStruct((N, D), jnp.float32),
            jax.ShapeDtypeStruct((N, D), jnp.float32),
        ],
    )(nodes, coords8, hn_W1, hn_b1.reshape(1, D), hn_W2, hn_b2.reshape(1, D),
      hc_W1p, hc_b1.reshape(1, D), hc_W2, hc_b2.reshape(1, D))


def _pack_words(even, odd):
    be = lax.bitcast_convert_type(even.astype(jnp.bfloat16).astype(jnp.float32),
                                  jnp.uint32)
    bo = lax.bitcast_convert_type(odd.astype(jnp.bfloat16).astype(jnp.float32),
                                  jnp.uint32)
    w = (bo & jnp.uint32(0xFFFF0000)) | (be >> 16)
    return lax.bitcast_convert_type(w, jnp.float32)


def _unpack_words(w):
    wi = lax.bitcast_convert_type(w, jnp.uint32)
    even = lax.bitcast_convert_type(wi << 16, jnp.float32)
    odd = lax.bitcast_convert_type(wi & jnp.uint32(0xFFFF0000), jnp.float32)
    return even, odd


def _st_body(h_ref, hc_ref, wae, wao, wbe, wbo, wce, wco, b1e, b1o,
             s_ref, t_ref):
    h = h_ref[...]
    hc = hc_ref[...]
    hce = jnp.dot(hc, wce[...], preferred_element_type=jnp.float32)
    hco = jnp.dot(hc, wco[...], preferred_element_type=jnp.float32)
    se = jnp.dot(h, wae[...], preferred_element_type=jnp.float32) + hce + b1e[...]
    so = jnp.dot(h, wao[...], preferred_element_type=jnp.float32) + hco + b1o[...]
    te = jnp.dot(h, wbe[...], preferred_element_type=jnp.float32) - hce
    to = jnp.dot(h, wbo[...], preferred_element_type=jnp.float32) - hco
    s_ref[...] = _pack_words(se, so)
    t_ref[...] = _pack_words(te, to)


def _node_transform(h, hc, wae, wao, wbe, wbo, wce, wco, b1e, b1o):
    hd = D // 2
    full = lambda: pl.BlockSpec((D, hd), lambda i: (0, 0))
    return pl.pallas_call(
        _st_body,
        grid=(NT_N,),
        in_specs=[
            pl.BlockSpec((TN, D), lambda i: (i, 0)),
            pl.BlockSpec((TN, D), lambda i: (i, 0)),
            full(), full(), full(), full(), full(), full(),
            pl.BlockSpec((1, hd), lambda i: (0, 0)),
            pl.BlockSpec((1, hd), lambda i: (0, 0)),
        ],
        out_specs=[
            pl.BlockSpec((TN, hd), lambda i: (i, 0)),
            pl.BlockSpec((TN, hd), lambda i: (i, 0)),
        ],
        out_shape=[
            jax.ShapeDtypeStruct((N, hd), jnp.float32),
            jax.ShapeDtypeStruct((N, hd), jnp.float32),
        ],
    )(h, hc, wae, wao, wbe, wbo, wce, wco,
      b1e.reshape(1, hd), b1o.reshape(1, hd))


_CH = 200


def _gather2(Sv, Tv, src, dst):
    info = plsc.get_sparse_core_info()
    nc, ns = info.num_cores, info.num_subcores
    nw = nc * ns
    epw = E // nw
    nch = epw // _CH
    hd = D // 2
    mesh = plsc.VectorSubcoreMesh(core_axis_name="c", subcore_axis_name="s")

    @functools.partial(
        pl.kernel,
        out_type=(jax.ShapeDtypeStruct((E, hd), jnp.float32),
                  jax.ShapeDtypeStruct((E, hd), jnp.float32)),
        mesh=mesh,
        scratch_types=[
            pltpu.VMEM((_CH,), jnp.int32), pltpu.VMEM((_CH,), jnp.int32),
            pltpu.VMEM((_CH,), jnp.int32), pltpu.VMEM((_CH,), jnp.int32),
            pltpu.VMEM((_CH, hd), jnp.float32), pltpu.VMEM((_CH, hd), jnp.float32),
            pltpu.VMEM((_CH, hd), jnp.float32), pltpu.VMEM((_CH, hd), jnp.float32),
            pltpu.SemaphoreType.DMA, pltpu.SemaphoreType.DMA,
            pltpu.SemaphoreType.DMA, pltpu.SemaphoreType.DMA,
            pltpu.SemaphoreType.DMA, pltpu.SemaphoreType.DMA,
        ],
    )
    def k(S_hbm, T_hbm, src_hbm, dst_hbm, P_hbm, Q_hbm,
          si0, si1, di0, di1, sr0, sr1, dr0, dr1,
          semi0, semi1, semg0, semg1, semo0, semo1):
        wid = lax.axis_index("s") * nc + lax.axis_index("c")
        base_w = wid * epw
        si = (si0, si1); di = (di0, di1)
        sr = (sr0, sr1); dr = (dr0, dr1)
        semi = (semi0, semi1); semg = (semg0, semg1); semo = (semo0, semo1)

        def start_idx(c, sl):
            base = base_w + c * _CH
            return (pltpu.async_copy(src_hbm.at[pl.ds(base, _CH)], si[sl], semi[sl]),
                    pltpu.async_copy(dst_hbm.at[pl.ds(base, _CH)], di[sl], semi[sl]))

        def start_gather(sl):
            return (pltpu.async_copy(S_hbm.at[si[sl]], sr[sl], semg[sl]),
                    pltpu.async_copy(T_hbm.at[di[sl]], dr[sl], semg[sl]))

        def start_out(c, sl):
            base = base_w + c * _CH
            return (pltpu.async_copy(sr[sl], P_hbm.at[pl.ds(base, _CH)], semo[sl]),
                    pltpu.async_copy(dr[sl], Q_hbm.at[pl.ds(base, _CH)], semo[sl]))

        g = {}; o = {}; idx = {}
        idx[0] = start_idx(0, 0)
        for cp in idx[0]:
            cp.wait()
        g[0] = start_gather(0)
        if nch > 1:
            idx[1] = start_idx(1, 1)
        for c in range(nch):
            sl = c & 1
            if c + 1 < nch:
                for cp in idx[c + 1]:
                    cp.wait()
                if c - 1 >= 0:
                    for cp in o[c - 1]:
                        cp.wait()
                g[c + 1] = start_gather(1 - sl)
            for cp in g[c]:
                cp.wait()
            o[c] = start_out(c, sl)
            if c + 2 < nch:
                idx[c + 2] = start_idx(c + 2, sl)
        for cc in (nch - 2, nch - 1):
            if cc >= 0 and cc in o:
                for cp in o[cc]:
                    cp.wait()

    return k(Sv, Tv, src, dst)


def _edge_body(p_ref, q_ref, w2ee, w2eo, w2oe, w2oo, b2e, b2o, m_ref):
    pe, po = _unpack_words(p_ref[...])
    qe, qo = _unpack_words(q_ref[...])
    ae = jnp.maximum(pe + qe, 0.0).astype(jnp.bfloat16)
    ao = jnp.maximum(po + qo, 0.0).astype(jnp.bfloat16)
    me = (jnp.dot(ae, w2ee[...], preferred_element_type=jnp.float32)
          + jnp.dot(ao, w2oe[...], preferred_element_type=jnp.float32) + b2e[...])
    mo = (jnp.dot(ae, w2eo[...], preferred_element_type=jnp.float32)
          + jnp.dot(ao, w2oo[...], preferred_element_type=jnp.float32) + b2o[...])
    m_ref[...] = _pack_words(me, mo)


def _edge_mlp(P, Q, w2ee, w2eo, w2oe, w2oo, b2e, b2o):
    hd = D // 2
    wspec = lambda: pl.BlockSpec((hd, hd), lambda i: (0, 0))
    return pl.pallas_call(
        _edge_body,
        grid=(NT_E,),
        in_specs=[
            pl.BlockSpec((TE, hd), lambda i: (i, 0)),
            pl.BlockSpec((TE, hd), lambda i: (i, 0)),
            wspec(), wspec(), wspec(), wspec(),
            pl.BlockSpec((1, hd), lambda i: (0, 0)),
            pl.BlockSpec((1, hd), lambda i: (0, 0)),
        ],
        out_specs=pl.BlockSpec((TE, hd), lambda i: (i, 0)),
        out_shape=jax.ShapeDtypeStruct((E, hd), jnp.float32),
    )(P, Q, w2ee, w2eo, w2oe, w2oo, b2e.reshape(1, hd), b2o.reshape(1, hd))


_WR = 320
_SCH = 2000
_CB = 64
_RING = 4096
_FL = 2048
_PLN = E + 2 * _FL
_PAD_PK = _WR


def _sc_mesh_info():
    info = plsc.get_sparse_core_info()
    return info.num_cores, info.num_subcores


def _scatter_plan(dst):
    nc, ns = _sc_mesh_info()
    nw = nc * ns
    nchunks = E // _SCH
    mesh = plsc.VectorSubcoreMesh(core_axis_name="c", subcore_axis_name="s")

    @functools.partial(
        pl.kernel,
        out_type=(jax.ShapeDtypeStruct((nw * _PLN,), jnp.int32),
                  jax.ShapeDtypeStruct((nw * 8,), jnp.int32)),
        mesh=mesh,
        compiler_params=pltpu.CompilerParams(needs_layout_passes=False),
        scratch_types=[
            pltpu.VMEM((_RING,), jnp.int32),
            pltpu.VMEM((_SCH,), jnp.int32),
            pltpu.VMEM((16,), jnp.int32),
        ],
    )
    def k(dst_hbm, plan_hbm, cnt_hbm, ring, dstv, cbuf):
        wid = lax.axis_index("s") * nc + lax.axis_index("c")
        lo = wid * _WR
        base_out = wid * _PLN
        iota = lax.iota(jnp.int32, 16)

        def flush(flushed):
            fl8 = pl.multiple_of(flushed, _FL)
            half = (fl8 >> 11) & 1
            pltpu.sync_copy(ring.at[pl.ds(half * _FL, _FL)],
                            plan_hbm.at[pl.ds(base_out + fl8, _FL)])
            return flushed + _FL

        def chunk(c, carry):
            cntv, flushed = carry
            pltpu.sync_copy(dst_hbm.at[pl.ds(c * _SCH, _SCH)], dstv)

            def scan_g(g, cntv):
                ld = dstv[pl.ds(g * 16, 16)] - lo
                msk = (ld >= 0) & (ld < _WR)
                pk = ((c * _SCH + g * 16 + iota) << 9) | (ld & 511)
                pos = (cntv + plsc.cumsum(msk.astype(jnp.int32)) - 1) & (_RING - 1)
                plsc.store_scatter(ring, [pos], pk, mask=msk)
                return cntv + plsc.all_reduce_population_count(msk)

            cntv = lax.fori_loop(0, _SCH // 16, scan_g, cntv)
            cnt_s = lax.reduce_max(cntv, (0,))

            def maybe_flush(flushed):
                return lax.cond(cnt_s - flushed >= _FL, flush,
                                lambda f: f, flushed)

            return cntv, maybe_flush(flushed)

        cntv, flushed = lax.fori_loop(
            0, nchunks, chunk,
            (jnp.zeros((16,), jnp.int32), jnp.int32(0)))
        cnt_s = lax.reduce_max(cntv, (0,))
        padv = jnp.full((16,), _PAD_PK, jnp.int32)
        for kk in range(4):
            plsc.store_scatter(ring, [(cnt_s + kk * 16 + iota) & (_RING - 1)], padv)

        def cond(fl):
            return fl < cnt_s + _CB

        flushed = lax.while_loop(cond, flush, flushed)
        cbuf[pl.ds(0, 16)] = cntv
        pltpu.sync_copy(cbuf.at[pl.ds(0, 8)], cnt_hbm.at[pl.ds(wid * 8, 8)])

    return k(dst)


def _scatter_max2(Mv, plan, counts):
    nc, ns = _sc_mesh_info()
    nw = nc * ns
    hd = D // 2
    mesh = plsc.VectorSubcoreMesh(core_axis_name="c", subcore_axis_name="s")

    @functools.partial(
        pl.kernel,
        out_type=jax.ShapeDtypeStruct((nw * _WR * D,), jnp.float32),
        mesh=mesh,
        compiler_params=pltpu.CompilerParams(needs_layout_passes=False),
        scratch_types=(
            [pltpu.VMEM(((_WR + 1) * 16,), jnp.float32) for _ in range(16)]
            + [
                pltpu.VMEM((_CB,), jnp.int32),
                pltpu.VMEM((_CB,), jnp.int32),
                pltpu.VMEM((_CB,), jnp.int32),
                pltpu.VMEM((_CB,), jnp.int32),
                pltpu.VMEM((_CB,), jnp.int32),
                pltpu.VMEM((_CB,), jnp.int32),
                pltpu.VMEM((_CB, hd), jnp.float32),
                pltpu.VMEM((_CB, hd), jnp.float32),
                pltpu.VMEM((16,), jnp.int32),
                pltpu.VMEM((40 * D,), jnp.float32),
                pltpu.SemaphoreType.DMA,
                pltpu.SemaphoreType.DMA,
            ]
        ),
    )
    def k(M_hbm, plan_hbm, cnt_hbm, agg_hbm, *rest):
        accs = rest[:16]
        (pke0, pke1, mid0, mid1, mld0, mld1, rows0, rows1, cbuf, stg,
         sem0, sem1) = rest[16:]
        wid = lax.axis_index("s") * nc + lax.axis_index("c")
        lo = wid * _WR
        base_in = wid * _PLN
        iota = lax.iota(jnp.int32, 16)
        neg = jnp.full((16,), _NEG_INF, jnp.float32)
        m16 = jnp.full((16,), -65536, jnp.int32)

        def initacc(i, _):
            for b in range(16):
                plsc.store_scatter(accs[b], [jnp.full((16,), i * 16, jnp.int32) + iota], neg)
            return 0
        lax.fori_loop(0, _WR + 1, initacc, 0)

        cbuf[pl.ds(0, 16)] = jnp.zeros((16,), jnp.int32)
        pltpu.sync_copy(cnt_hbm.at[pl.ds(wid * 8, 8)], cbuf.at[pl.ds(0, 8)])
        cnt = lax.reduce_max(cbuf[pl.ds(0, 16)], (0,))
        nbat = (cnt + _CB - 1) // _CB

        slots = ((pke0, mid0, mld0, rows0, sem0),
                 (pke1, mid1, mld1, rows1, sem1))

        def stage(b, slot):
            pke, mid, mld, rows, sem = slots[slot]
            pltpu.sync_copy(plan_hbm.at[pl.ds(base_in + b * _CB, _CB)], pke)
            for i in range(_CB // 16):
                pk = pke[pl.ds(i * 16, 16)]
                mid[pl.ds(i * 16, 16)] = pk >> 9
                mld[pl.ds(i * 16, 16)] = pk & 511
            return pltpu.async_copy(M_hbm.at[mid], rows, sem)

        def process(slot):
            _, _, mld, rows, _ = slots[slot]

            def row(r, _):
                rfull = jnp.full((16,), r, jnp.int32)
                lds = plsc.load_gather(mld, [rfull])
                lds = jnp.minimum(lds, _WR)
                for j in range(hd // 16):
                    w = plsc.load_gather(rows, [rfull, iota + j * 16])
                    wi = plsc.bitcast(w, jnp.int32)
                    lov = plsc.bitcast(wi << 16, jnp.float32)
                    hiv = plsc.bitcast(wi & m16, jnp.float32)
                    addr = lds * 16 + iota
                    for par, val in ((0, lov), (1, hiv)):
                        ref = accs[2 * j + par]
                        a = plsc.load_gather(ref, [addr])
                        plsc.store_scatter(ref, [addr], jnp.maximum(a, val))
                return 0

            lax.fori_loop(0, _CB, row, 0)

        @pl.when(nbat > 0)
        def _run():
            stage(0, 0)

            def pair(i, _):
                b1 = 2 * i + 1

                @pl.when(b1 < nbat)
                def _s1():
                    stage(b1, 1)

                pltpu.make_async_copy(M_hbm.at[slots[0][1]], slots[0][3],
                                      slots[0][4]).wait()
                process(0)

                @pl.when(b1 + 1 < nbat)
                def _s0():
                    stage(b1 + 1, 0)

                @pl.when(b1 < nbat)
                def _p1():
                    pltpu.make_async_copy(M_hbm.at[slots[1][1]], slots[1][3],
                                          slots[1][4]).wait()
                    process(1)
                return 0

            lax.fori_loop(0, (nbat + 1) // 2, pair, 0)

        def rowchunk(rc, _):
            def rowcp(r2, _):
                rowv = jnp.full((16,), (rc * 40 + r2) * 16, jnp.int32) + iota
                for j in range(8):
                    for par in range(2):
                        v = plsc.load_gather(accs[2 * j + par], [rowv])
                        plsc.store_scatter(
                            stg, [r2 * D + 32 * j + 2 * iota + par], v)
                return 0

            lax.fori_loop(0, 40, rowcp, 0)
            pltpu.sync_copy(
                stg, agg_hbm.at[pl.ds((lo + rc * 40) * D, 40 * D)])
            return 0

        lax.fori_loop(0, _WR // 40, rowchunk, 0)

    return k(Mv, plan, counts)


def _upd_body(h_ref, agg_ref, out_ref):
    agg = agg_ref[...]
    out_ref[...] = h_ref[...] + jnp.where(jnp.isneginf(agg), 0.0, agg)


def _h_update(h, agg):
    return pl.pallas_call(
        _upd_body,
        grid=(NT_N,),
        in_specs=[
            pl.BlockSpec((TN, D), lambda i: (i, 0)),
            pl.BlockSpec((TN, D), lambda i: (i, 0)),
        ],
        out_specs=pl.BlockSpec((TN, D), lambda i: (i, 0)),
        out_shape=jax.ShapeDtypeStruct((N, D), jnp.float32),
    )(h, agg)


def kernel(nodes, coords, edge_index, hn_W1, hn_b1, hn_W2, hn_b2,
           hc_W1, hc_b1, hc_W2, hc_b2, mp_W1, mp_b1, mp_W2, mp_b2):
    src = edge_index[0]
    dst = edge_index[1]
    coords8 = jnp.pad(coords, ((0, 0), (0, 5)))
    hc_W1p = jnp.pad(hc_W1, ((0, 5), (0, 0)))

    h, hcv = _encode(nodes, coords8, hn_W1, hn_b1, hn_W2, hn_b2,
                     hc_W1p, hc_b1, hc_W2, hc_b2)

    plan, counts = _scatter_plan(dst)

    for i in range(3):
        wa = mp_W1[i, 0:D, :]
        wb = mp_W1[i, D:2 * D, :]
        wc = mp_W1[i, 2 * D:3 * D, :]
        w2 = mp_W2[i]
        w2b = w2.astype(jnp.bfloat16)
        S, T = _node_transform(h, hcv,
                               wa[:, 0::2], wa[:, 1::2],
                               wb[:, 0::2], wb[:, 1::2],
                               wc[:, 0::2], wc[:, 1::2],
                               mp_b1[i, 0::2], mp_b1[i, 1::2])
        P, Q = _gather2(S, T, src, dst)
        M = _edge_mlp(P, Q,
                      w2b[0::2, 0::2], w2b[0::2, 1::2],
                      w2b[1::2, 0::2], w2b[1::2, 1::2],
                      mp_b2[i, 0::2], mp_b2[i, 1::2])
        agg = _scatter_max2(M, plan, counts).reshape(-1, D)[:N]
        h = _h_update(h, agg)
    return h


# speedup vs baseline: 3.7553x; 1.0744x over previous
import functools

import jax
import jax.numpy as jnp
from jax import lax
from jax.experimental import pallas as pl
from jax.experimental.pallas import tpu as pltpu
from jax.experimental.pallas import tpu_sc as plsc

N = 10000
E = 160000
D = 256
NT_N = 10
TN = N // NT_N
NT_E = 160
TE = E // NT_E

_NEG_INF = float("-inf")


def _enc_body(nodes_ref, coords_ref, w1n, b1n, w2n, b2n, w1c, b1c, w2c, b2c,
              h_ref, hc_ref):
    t = jnp.maximum(
        jnp.dot(nodes_ref[...], w1n[...], preferred_element_type=jnp.float32)
        + b1n[...], 0.0)
    h_ref[...] = jnp.dot(t, w2n[...], preferred_element_type=jnp.float32) + b2n[...]
    t2 = jnp.maximum(
        jnp.dot(coords_ref[...], w1c[...], preferred_element_type=jnp.float32)
        + b1c[...], 0.0)
    hc_ref[...] = jnp.dot(t2, w2c[...], preferred_element_type=jnp.float32) + b2c[...]


def _encode(nodes, coords8, hn_W1, hn_b1, hn_W2, hn_b2, hc_W1p, hc_b1, hc_W2, hc_b2):
    full = lambda shape: pl.BlockSpec(shape, lambda i: (0, 0))
    return pl.pallas_call(
        _enc_body,
        grid=(NT_N,),
        in_specs=[
            pl.BlockSpec((TN, 128), lambda i: (i, 0)),
            pl.BlockSpec((TN, 8), lambda i: (i, 0)),
            full((128, D)), full((1, D)), full((D, D)), full((1, D)),
            full((8, D)), full((1, D)), full((D, D)), full((1, D)),
        ],
        out_specs=[
            pl.BlockSpec((TN, D), lambda i: (i, 0)),
            pl.BlockSpec((TN, D), lambda i: (i, 0)),
        ],
        out_shape=[
            jax.ShapeDtypeStruct((N, D), jnp.float32),
            jax.ShapeDtypeStruct((N, D), jnp.float32),
        ],
    )(nodes, coords8, hn_W1, hn_b1.reshape(1, D), hn_W2, hn_b2.reshape(1, D),
      hc_W1p, hc_b1.reshape(1, D), hc_W2, hc_b2.reshape(1, D))


def _pack_words(even, odd):
    be = lax.bitcast_convert_type(even.astype(jnp.bfloat16).astype(jnp.float32),
                                  jnp.uint32)
    bo = lax.bitcast_convert_type(odd.astype(jnp.bfloat16).astype(jnp.float32),
                                  jnp.uint32)
    w = (bo & jnp.uint32(0xFFFF0000)) | (be >> 16)
    return lax.bitcast_convert_type(w, jnp.float32)


def _unpack_words(w):
    wi = lax.bitcast_convert_type(w, jnp.uint32)
    even = lax.bitcast_convert_type(wi << 16, jnp.float32)
    odd = lax.bitcast_convert_type(wi & jnp.uint32(0xFFFF0000), jnp.float32)
    return even, odd


def _st_body(h_ref, hc_ref, wae, wao, wbe, wbo, wce, wco, b1e, b1o,
             s_ref, t_ref):
    h = h_ref[...]
    hc = hc_ref[...]
    hce = jnp.dot(hc, wce[...], preferred_element_type=jnp.float32)
    hco = jnp.dot(hc, wco[...], preferred_element_type=jnp.float32)
    se = jnp.dot(h, wae[...], preferred_element_type=jnp.float32) + hce + b1e[...]
    so = jnp.dot(h, wao[...], preferred_element_type=jnp.float32) + hco + b1o[...]
    te = jnp.dot(h, wbe[...], preferred_element_type=jnp.float32) - hce
    to = jnp.dot(h, wbo[...], preferred_element_type=jnp.float32) - hco
    s_ref[...] = _pack_words(se, so)
    t_ref[...] = _pack_words(te, to)


def _node_transform(h, hc, wae, wao, wbe, wbo, wce, wco, b1e, b1o):
    hd = D // 2
    full = lambda: pl.BlockSpec((D, hd), lambda i: (0, 0))
    return pl.pallas_call(
        _st_body,
        grid=(NT_N,),
        in_specs=[
            pl.BlockSpec((TN, D), lambda i: (i, 0)),
            pl.BlockSpec((TN, D), lambda i: (i, 0)),
            full(), full(), full(), full(), full(), full(),
            pl.BlockSpec((1, hd), lambda i: (0, 0)),
            pl.BlockSpec((1, hd), lambda i: (0, 0)),
        ],
        out_specs=[
            pl.BlockSpec((TN, hd), lambda i: (i, 0)),
            pl.BlockSpec((TN, hd), lambda i: (i, 0)),
        ],
        out_shape=[
            jax.ShapeDtypeStruct((N, hd), jnp.float32),
            jax.ShapeDtypeStruct((N, hd), jnp.float32),
        ],
    )(h, hc, wae, wao, wbe, wbo, wce, wco,
      b1e.reshape(1, hd), b1o.reshape(1, hd))


_CH = 200


def _gather2(Sv, Tv, src, dst):
    info = plsc.get_sparse_core_info()
    nc, ns = info.num_cores, info.num_subcores
    nw = nc * ns
    epw = E // nw
    nch = epw // _CH
    hd = D // 2
    mesh = plsc.VectorSubcoreMesh(core_axis_name="c", subcore_axis_name="s")

    @functools.partial(
        pl.kernel,
        out_type=(jax.ShapeDtypeStruct((E, hd), jnp.float32),
                  jax.ShapeDtypeStruct((E, hd), jnp.float32)),
        mesh=mesh,
        scratch_types=[
            pltpu.VMEM((_CH,), jnp.int32), pltpu.VMEM((_CH,), jnp.int32),
            pltpu.VMEM((_CH,), jnp.int32), pltpu.VMEM((_CH,), jnp.int32),
            pltpu.VMEM((_CH, hd), jnp.float32), pltpu.VMEM((_CH, hd), jnp.float32),
            pltpu.VMEM((_CH, hd), jnp.float32), pltpu.VMEM((_CH, hd), jnp.float32),
            pltpu.SemaphoreType.DMA, pltpu.SemaphoreType.DMA,
            pltpu.SemaphoreType.DMA, pltpu.SemaphoreType.DMA,
            pltpu.SemaphoreType.DMA, pltpu.SemaphoreType.DMA,
        ],
    )
    def k(S_hbm, T_hbm, src_hbm, dst_hbm, P_hbm, Q_hbm,
          si0, si1, di0, di1, sr0, sr1, dr0, dr1,
          semi0, semi1, semg0, semg1, semo0, semo1):
        wid = lax.axis_index("s") * nc + lax.axis_index("c")
        base_w = wid * epw
        si = (si0, si1); di = (di0, di1)
        sr = (sr0, sr1); dr = (dr0, dr1)
        semi = (semi0, semi1); semg = (semg0, semg1); semo = (semo0, semo1)

        def start_idx(c, sl):
            base = base_w + c * _CH
            return (pltpu.async_copy(src_hbm.at[pl.ds(base, _CH)], si[sl], semi[sl]),
                    pltpu.async_copy(dst_hbm.at[pl.ds(base, _CH)], di[sl], semi[sl]))

        def start_gather(sl):
            return (pltpu.async_copy(S_hbm.at[si[sl]], sr[sl], semg[sl]),
                    pltpu.async_copy(T_hbm.at[di[sl]], dr[sl], semg[sl]))

        def start_out(c, sl):
            base = base_w + c * _CH
            return (pltpu.async_copy(sr[sl], P_hbm.at[pl.ds(base, _CH)], semo[sl]),
                    pltpu.async_copy(dr[sl], Q_hbm.at[pl.ds(base, _CH)], semo[sl]))

        g = {}; o = {}; idx = {}
        idx[0] = start_idx(0, 0)
        for cp in idx[0]:
            cp.wait()
        g[0] = start_gather(0)
        if nch > 1:
            idx[1] = start_idx(1, 1)
        for c in range(nch):
            sl = c & 1
            if c + 1 < nch:
                for cp in idx[c + 1]:
                    cp.wait()
                if c - 1 >= 0:
                    for cp in o[c - 1]:
                        cp.wait()
                g[c + 1] = start_gather(1 - sl)
            for cp in g[c]:
                cp.wait()
            o[c] = start_out(c, sl)
            if c + 2 < nch:
                idx[c + 2] = start_idx(c + 2, sl)
        for cc in (nch - 2, nch - 1):
            if cc >= 0 and cc in o:
                for cp in o[cc]:
                    cp.wait()

    return k(Sv, Tv, src, dst)


def _edge_body(p_ref, q_ref, w2ee, w2eo, w2oe, w2oo, b2e, b2o, m_ref):
    pe, po = _unpack_words(p_ref[...])
    qe, qo = _unpack_words(q_ref[...])
    ae = jnp.maximum(pe + qe, 0.0).astype(jnp.bfloat16)
    ao = jnp.maximum(po + qo, 0.0).astype(jnp.bfloat16)
    me = (jnp.dot(ae, w2ee[...], preferred_element_type=jnp.float32)
          + jnp.dot(ao, w2oe[...], preferred_element_type=jnp.float32) + b2e[...])
    mo = (jnp.dot(ae, w2eo[...], preferred_element_type=jnp.float32)
          + jnp.dot(ao, w2oo[...], preferred_element_type=jnp.float32) + b2o[...])
    m_ref[...] = _pack_words(me, mo)


def _edge_mlp(P, Q, w2ee, w2eo, w2oe, w2oo, b2e, b2o):
    hd = D // 2
    wspec = lambda: pl.BlockSpec((hd, hd), lambda i: (0, 0))
    return pl.pallas_call(
        _edge_body,
        grid=(NT_E,),
        in_specs=[
            pl.BlockSpec((TE, hd), lambda i: (i, 0)),
            pl.BlockSpec((TE, hd), lambda i: (i, 0)),
            wspec(), wspec(), wspec(), wspec(),
            pl.BlockSpec((1, hd), lambda i: (0, 0)),
            pl.BlockSpec((1, hd), lambda i: (0, 0)),
        ],
        out_specs=pl.BlockSpec((TE, hd), lambda i: (i, 0)),
        out_shape=jax.ShapeDtypeStruct((E, hd), jnp.float32),
    )(P, Q, w2ee, w2eo, w2oe, w2oo, b2e.reshape(1, hd), b2o.reshape(1, hd))


_WR = 320
_SCH = 2000
_CB = 64
_RING = 4096
_FL = 2048
_PLN = E + 2 * _FL
_PAD_PK = _WR


def _sc_mesh_info():
    info = plsc.get_sparse_core_info()
    return info.num_cores, info.num_subcores


def _scatter_plan(dst):
    nc, ns = _sc_mesh_info()
    nw = nc * ns
    nchunks = E // _SCH
    mesh = plsc.VectorSubcoreMesh(core_axis_name="c", subcore_axis_name="s")

    @functools.partial(
        pl.kernel,
        out_type=(jax.ShapeDtypeStruct((nw * _PLN,), jnp.int32),
                  jax.ShapeDtypeStruct((nw * 8,), jnp.int32)),
        mesh=mesh,
        compiler_params=pltpu.CompilerParams(needs_layout_passes=False),
        scratch_types=[
            pltpu.VMEM((_RING,), jnp.int32),
            pltpu.VMEM((_SCH,), jnp.int32),
            pltpu.VMEM((16,), jnp.int32),
        ],
    )
    def k(dst_hbm, plan_hbm, cnt_hbm, ring, dstv, cbuf):
        wid = lax.axis_index("s") * nc + lax.axis_index("c")
        lo = wid * _WR
        base_out = wid * _PLN
        iota = lax.iota(jnp.int32, 16)

        def flush(flushed):
            fl8 = pl.multiple_of(flushed, _FL)
            half = (fl8 >> 11) & 1
            pltpu.sync_copy(ring.at[pl.ds(half * _FL, _FL)],
                            plan_hbm.at[pl.ds(base_out + fl8, _FL)])
            return flushed + _FL

        def chunk(c, carry):
            cntv, flushed = carry
            pltpu.sync_copy(dst_hbm.at[pl.ds(c * _SCH, _SCH)], dstv)

            def scan_g(g, cntv):
                ld = dstv[pl.ds(g * 16, 16)] - lo
                msk = (ld >= 0) & (ld < _WR)
                pk = ((c * _SCH + g * 16 + iota) << 9) | (ld & 511)
                pos = (cntv + plsc.cumsum(msk.astype(jnp.int32)) - 1) & (_RING - 1)
                plsc.store_scatter(ring, [pos], pk, mask=msk)
                return cntv + plsc.all_reduce_population_count(msk)

            cntv = lax.fori_loop(0, _SCH // 16, scan_g, cntv)
            cnt_s = lax.reduce_max(cntv, (0,))

            def maybe_flush(flushed):
                return lax.cond(cnt_s - flushed >= _FL, flush,
                                lambda f: f, flushed)

            return cntv, maybe_flush(flushed)

        cntv, flushed = lax.fori_loop(
            0, nchunks, chunk,
            (jnp.zeros((16,), jnp.int32), jnp.int32(0)))
        cnt_s = lax.reduce_max(cntv, (0,))
        padv = jnp.full((16,), _PAD_PK, jnp.int32)
        for kk in range(4):
            plsc.store_scatter(ring, [(cnt_s + kk * 16 + iota) & (_RING - 1)], padv)

        def cond(fl):
            return fl < cnt_s + _CB

        flushed = lax.while_loop(cond, flush, flushed)
        cbuf[pl.ds(0, 16)] = cntv
        pltpu.sync_copy(cbuf.at[pl.ds(0, 8)], cnt_hbm.at[pl.ds(wid * 8, 8)])

    return k(dst)


def _scatter_max2(Mv, plan, counts):
    nc, ns = _sc_mesh_info()
    nw = nc * ns
    hd = D // 2
    mesh = plsc.VectorSubcoreMesh(core_axis_name="c", subcore_axis_name="s")

    @functools.partial(
        pl.kernel,
        out_type=jax.ShapeDtypeStruct((nw * _WR * D,), jnp.float32),
        mesh=mesh,
        compiler_params=pltpu.CompilerParams(needs_layout_passes=False),
        scratch_types=(
            [pltpu.VMEM(((_WR + 1) * 16,), jnp.float32) for _ in range(16)]
            + [
                pltpu.VMEM((16 * _CB,), jnp.int32),
                pltpu.VMEM((_CB,), jnp.int32),
                pltpu.VMEM((_CB,), jnp.int32),
                pltpu.VMEM((_CB,), jnp.int32),
                pltpu.VMEM((_CB,), jnp.int32),
                pltpu.VMEM((_CB, hd), jnp.float32),
                pltpu.VMEM((_CB, hd), jnp.float32),
                pltpu.VMEM((16,), jnp.int32),
                pltpu.VMEM((40 * D,), jnp.float32),
                pltpu.SemaphoreType.DMA,
                pltpu.SemaphoreType.DMA,
            ]
        ),
    )
    def k(M_hbm, plan_hbm, cnt_hbm, agg_hbm, *rest):
        accs = rest[:16]
        (pkeb, mid0, mid1, mld0, mld1, rows0, rows1, cbuf, stg,
         sem0, sem1) = rest[16:]
        wid = lax.axis_index("s") * nc + lax.axis_index("c")
        lo = wid * _WR
        base_in = wid * _PLN
        iota = lax.iota(jnp.int32, 16)
        neg = jnp.full((16,), _NEG_INF, jnp.float32)
        m16 = jnp.full((16,), -65536, jnp.int32)

        def initacc(i, _):
            for b in range(16):
                plsc.store_scatter(accs[b], [jnp.full((16,), i * 16, jnp.int32) + iota], neg)
            return 0
        lax.fori_loop(0, _WR + 1, initacc, 0)

        cbuf[pl.ds(0, 16)] = jnp.zeros((16,), jnp.int32)
        pltpu.sync_copy(cnt_hbm.at[pl.ds(wid * 8, 8)], cbuf.at[pl.ds(0, 8)])
        cnt = lax.reduce_max(cbuf[pl.ds(0, 16)], (0,))
        nbat = (cnt + _CB - 1) // _CB

        slots = ((mid0, mld0, rows0, sem0),
                 (mid1, mld1, rows1, sem1))

        def fetch_sb(sb):
            off = pl.multiple_of(sb * 16 * _CB, 16 * _CB)
            pltpu.sync_copy(plan_hbm.at[pl.ds(base_in + off, 16 * _CB)], pkeb)

        def stage(b, slot):
            mid, mld, rows, sem = slots[slot]
            lb = pl.multiple_of((b & 15) * _CB, _CB)
            for i in range(_CB // 16):
                pk = pkeb[pl.ds(lb + i * 16, 16)]
                mid[pl.ds(i * 16, 16)] = pk >> 9
                mld[pl.ds(i * 16, 16)] = pk & 511
            return pltpu.async_copy(M_hbm.at[mid], rows, sem)

        def process(slot):
            mld, rows = slots[slot][1], slots[slot][2]

            def group(g, _):
                mldv = jnp.minimum(mld[pl.ds(g * 16, 16)], _WR)

                def row(r2, _):
                    r = g * 16 + r2
                    rfull = jnp.full((16,), r, jnp.int32)
                    lds = jnp.take(mldv, jnp.full((16,), r2, jnp.int32))
                    addr = lds * 16 + iota
                    for j in range(hd // 16):
                        w = plsc.load_gather(rows, [rfull, iota + j * 16])
                        wi = plsc.bitcast(w, jnp.int32)
                        lov = plsc.bitcast(wi << 16, jnp.float32)
                        hiv = plsc.bitcast(wi & m16, jnp.float32)
                        for par, val in ((0, lov), (1, hiv)):
                            ref = accs[2 * j + par]
                            a = plsc.load_gather(ref, [addr])
                            plsc.store_scatter(ref, [addr], jnp.maximum(a, val))
                    return 0

                lax.fori_loop(0, 16, row, 0)
                return 0

            lax.fori_loop(0, _CB // 16, group, 0)

        @pl.when(nbat > 0)
        def _run():
            fetch_sb(0)
            stage(0, 0)

            def pair(i, _):
                b1 = 2 * i + 1

                @pl.when(b1 < nbat)
                def _s1():
                    stage(b1, 1)

                pltpu.make_async_copy(M_hbm.at[slots[0][0]], slots[0][2],
                                      slots[0][3]).wait()
                process(0)

                @pl.when(b1 + 1 < nbat)
                def _s0():
                    @pl.when(((b1 + 1) & 15) == 0)
                    def _f():
                        fetch_sb((b1 + 1) >> 4)
                    stage(b1 + 1, 0)

                @pl.when(b1 < nbat)
                def _p1():
                    pltpu.make_async_copy(M_hbm.at[slots[1][0]], slots[1][2],
                                          slots[1][3]).wait()
                    process(1)
                return 0

            lax.fori_loop(0, (nbat + 1) // 2, pair, 0)

        def rowchunk(rc, _):
            def rowcp(r2, _):
                rowv = jnp.full((16,), (rc * 40 + r2) * 16, jnp.int32) + iota
                for j in range(8):
                    for par in range(2):
                        v = plsc.load_gather(accs[2 * j + par], [rowv])
                        plsc.store_scatter(
                            stg, [r2 * D + 32 * j + 2 * iota + par], v)
                return 0

            lax.fori_loop(0, 40, rowcp, 0)
            pltpu.sync_copy(
                stg, agg_hbm.at[pl.ds((lo + rc * 40) * D, 40 * D)])
            return 0

        lax.fori_loop(0, _WR // 40, rowchunk, 0)

    return k(Mv, plan, counts)


def _upd_body(h_ref, agg_ref, out_ref):
    agg = agg_ref[...]
    out_ref[...] = h_ref[...] + jnp.where(jnp.isneginf(agg), 0.0, agg)


def _h_update(h, agg):
    return pl.pallas_call(
        _upd_body,
        grid=(NT_N,),
        in_specs=[
            pl.BlockSpec((TN, D), lambda i: (i, 0)),
            pl.BlockSpec((TN, D), lambda i: (i, 0)),
        ],
        out_specs=pl.BlockSpec((TN, D), lambda i: (i, 0)),
        out_shape=jax.ShapeDtypeStruct((N, D), jnp.float32),
    )(h, agg)


def kernel(nodes, coords, edge_index, hn_W1, hn_b1, hn_W2, hn_b2,
           hc_W1, hc_b1, hc_W2, hc_b2, mp_W1, mp_b1, mp_W2, mp_b2):
    src = edge_index[0]
    dst = edge_index[1]
    coords8 = jnp.pad(coords, ((0, 0), (0, 5)))
    hc_W1p = jnp.pad(hc_W1, ((0, 5), (0, 0)))

    h, hcv = _encode(nodes, coords8, hn_W1, hn_b1, hn_W2, hn_b2,
                     hc_W1p, hc_b1, hc_W2, hc_b2)

    plan, counts = _scatter_plan(dst)

    for i in range(3):
        wa = mp_W1[i, 0:D, :]
        wb = mp_W1[i, D:2 * D, :]
        wc = mp_W1[i, 2 * D:3 * D, :]
        w2 = mp_W2[i]
        w2b = w2.astype(jnp.bfloat16)
        S, T = _node_transform(h, hcv,
                               wa[:, 0::2], wa[:, 1::2],
                               wb[:, 0::2], wb[:, 1::2],
                               wc[:, 0::2], wc[:, 1::2],
                               mp_b1[i, 0::2], mp_b1[i, 1::2])
        P, Q = _gather2(S, T, src, dst)
        M = _edge_mlp(P, Q,
                      w2b[0::2, 0::2], w2b[0::2, 1::2],
                      w2b[1::2, 0::2], w2b[1::2, 1::2],
                      mp_b2[i, 0::2], mp_b2[i, 1::2])
        agg = _scatter_max2(M, plan, counts).reshape(-1, D)[:N]
        h = _h_update(h, agg)
    return h


# speedup vs baseline: 3.7800x; 1.0066x over previous
import functools

import jax
import jax.numpy as jnp
from jax import lax
from jax.experimental import pallas as pl
from jax.experimental.pallas import tpu as pltpu
from jax.experimental.pallas import tpu_sc as plsc

N = 10000
E = 160000
D = 256
NT_N = 10
TN = N // NT_N
NT_E = 160
TE = E // NT_E

_NEG_INF = float("-inf")


def _enc_body(nodes_ref, coords_ref, w1n, b1n, w2n, b2n, w1c, b1c, w2c, b2c,
              h_ref, hc_ref):
    t = jnp.maximum(
        jnp.dot(nodes_ref[...], w1n[...], preferred_element_type=jnp.float32)
        + b1n[...], 0.0)
    h_ref[...] = jnp.dot(t, w2n[...], preferred_element_type=jnp.float32) + b2n[...]
    t2 = jnp.maximum(
        jnp.dot(coords_ref[...], w1c[...], preferred_element_type=jnp.float32)
        + b1c[...], 0.0)
    hc_ref[...] = jnp.dot(t2, w2c[...], preferred_element_type=jnp.float32) + b2c[...]


def _encode(nodes, coords8, hn_W1, hn_b1, hn_W2, hn_b2, hc_W1p, hc_b1, hc_W2, hc_b2):
    full = lambda shape: pl.BlockSpec(shape, lambda i: (0, 0))
    return pl.pallas_call(
        _enc_body,
        grid=(NT_N,),
        in_specs=[
            pl.BlockSpec((TN, 128), lambda i: (i, 0)),
            pl.BlockSpec((TN, 8), lambda i: (i, 0)),
            full((128, D)), full((1, D)), full((D, D)), full((1, D)),
            full((8, D)), full((1, D)), full((D, D)), full((1, D)),
        ],
        out_specs=[
            pl.BlockSpec((TN, D), lambda i: (i, 0)),
            pl.BlockSpec((TN, D), lambda i: (i, 0)),
        ],
        out_shape=[
            jax.ShapeDtypeStruct((N, D), jnp.float32),
            jax.ShapeDtypeStruct((N, D), jnp.float32),
        ],
    )(nodes, coords8, hn_W1, hn_b1.reshape(1, D), hn_W2, hn_b2.reshape(1, D),
      hc_W1p, hc_b1.reshape(1, D), hc_W2, hc_b2.reshape(1, D))


def _pack_words(even, odd):
    be = lax.bitcast_convert_type(even.astype(jnp.bfloat16).astype(jnp.float32),
                                  jnp.uint32)
    bo = lax.bitcast_convert_type(odd.astype(jnp.bfloat16).astype(jnp.float32),
                                  jnp.uint32)
    w = (bo & jnp.uint32(0xFFFF0000)) | (be >> 16)
    return lax.bitcast_convert_type(w, jnp.float32)


def _unpack_words(w):
    wi = lax.bitcast_convert_type(w, jnp.uint32)
    even = lax.bitcast_convert_type(wi << 16, jnp.float32)
    odd = lax.bitcast_convert_type(wi & jnp.uint32(0xFFFF0000), jnp.float32)
    return even, odd


def _st_body(h_ref, hc_ref, wae, wao, wbe, wbo, wce, wco, b1e, b1o,
             s_ref, t_ref):
    h = h_ref[...]
    hc = hc_ref[...]
    hce = jnp.dot(hc, wce[...], preferred_element_type=jnp.float32)
    hco = jnp.dot(hc, wco[...], preferred_element_type=jnp.float32)
    se = jnp.dot(h, wae[...], preferred_element_type=jnp.float32) + hce + b1e[...]
    so = jnp.dot(h, wao[...], preferred_element_type=jnp.float32) + hco + b1o[...]
    te = jnp.dot(h, wbe[...], preferred_element_type=jnp.float32) - hce
    to = jnp.dot(h, wbo[...], preferred_element_type=jnp.float32) - hco
    s_ref[...] = _pack_words(se, so)
    t_ref[...] = _pack_words(te, to)


def _node_transform(h, hc, wae, wao, wbe, wbo, wce, wco, b1e, b1o):
    hd = D // 2
    full = lambda: pl.BlockSpec((D, hd), lambda i: (0, 0))
    return pl.pallas_call(
        _st_body,
        grid=(NT_N,),
        in_specs=[
            pl.BlockSpec((TN, D), lambda i: (i, 0)),
            pl.BlockSpec((TN, D), lambda i: (i, 0)),
            full(), full(), full(), full(), full(), full(),
            pl.BlockSpec((1, hd), lambda i: (0, 0)),
            pl.BlockSpec((1, hd), lambda i: (0, 0)),
        ],
        out_specs=[
            pl.BlockSpec((TN, hd), lambda i: (i, 0)),
            pl.BlockSpec((TN, hd), lambda i: (i, 0)),
        ],
        out_shape=[
            jax.ShapeDtypeStruct((N, hd), jnp.float32),
            jax.ShapeDtypeStruct((N, hd), jnp.float32),
        ],
    )(h, hc, wae, wao, wbe, wbo, wce, wco,
      b1e.reshape(1, hd), b1o.reshape(1, hd))


_CH = 200


def _gather2(Sv, Tv, src, dst):
    info = plsc.get_sparse_core_info()
    nc, ns = info.num_cores, info.num_subcores
    nw = nc * ns
    epw = E // nw
    nch = epw // _CH
    hd = D // 2
    mesh = plsc.VectorSubcoreMesh(core_axis_name="c", subcore_axis_name="s")

    @functools.partial(
        pl.kernel,
        out_type=(jax.ShapeDtypeStruct((E, hd), jnp.float32),
                  jax.ShapeDtypeStruct((E, hd), jnp.float32)),
        mesh=mesh,
        scratch_types=[
            pltpu.VMEM((_CH,), jnp.int32), pltpu.VMEM((_CH,), jnp.int32),
            pltpu.VMEM((_CH,), jnp.int32), pltpu.VMEM((_CH,), jnp.int32),
            pltpu.VMEM((_CH, hd), jnp.float32), pltpu.VMEM((_CH, hd), jnp.float32),
            pltpu.VMEM((_CH, hd), jnp.float32), pltpu.VMEM((_CH, hd), jnp.float32),
            pltpu.SemaphoreType.DMA, pltpu.SemaphoreType.DMA,
            pltpu.SemaphoreType.DMA, pltpu.SemaphoreType.DMA,
            pltpu.SemaphoreType.DMA, pltpu.SemaphoreType.DMA,
        ],
    )
    def k(S_hbm, T_hbm, src_hbm, dst_hbm, P_hbm, Q_hbm,
          si0, si1, di0, di1, sr0, sr1, dr0, dr1,
          semi0, semi1, semg0, semg1, semo0, semo1):
        wid = lax.axis_index("s") * nc + lax.axis_index("c")
        base_w = wid * epw
        si = (si0, si1); di = (di0, di1)
        sr = (sr0, sr1); dr = (dr0, dr1)
        semi = (semi0, semi1); semg = (semg0, semg1); semo = (semo0, semo1)

        def start_idx(c, sl):
            base = base_w + c * _CH
            return (pltpu.async_copy(src_hbm.at[pl.ds(base, _CH)], si[sl], semi[sl]),
                    pltpu.async_copy(dst_hbm.at[pl.ds(base, _CH)], di[sl], semi[sl]))

        def start_gather(sl):
            return (pltpu.async_copy(S_hbm.at[si[sl]], sr[sl], semg[sl]),
                    pltpu.async_copy(T_hbm.at[di[sl]], dr[sl], semg[sl]))

        def start_out(c, sl):
            base = base_w + c * _CH
            return (pltpu.async_copy(sr[sl], P_hbm.at[pl.ds(base, _CH)], semo[sl]),
                    pltpu.async_copy(dr[sl], Q_hbm.at[pl.ds(base, _CH)], semo[sl]))

        g = {}; o = {}; idx = {}
        idx[0] = start_idx(0, 0)
        for cp in idx[0]:
            cp.wait()
        g[0] = start_gather(0)
        if nch > 1:
            idx[1] = start_idx(1, 1)
        for c in range(nch):
            sl = c & 1
            if c + 1 < nch:
                for cp in idx[c + 1]:
                    cp.wait()
                if c - 1 >= 0:
                    for cp in o[c - 1]:
                        cp.wait()
                g[c + 1] = start_gather(1 - sl)
            for cp in g[c]:
                cp.wait()
            o[c] = start_out(c, sl)
            if c + 2 < nch:
                idx[c + 2] = start_idx(c + 2, sl)
        for cc in (nch - 2, nch - 1):
            if cc >= 0 and cc in o:
                for cp in o[cc]:
                    cp.wait()

    return k(Sv, Tv, src, dst)


def _edge_body(p_ref, q_ref, w2ee, w2eo, w2oe, w2oo, b2e, b2o, m_ref):
    pe, po = _unpack_words(p_ref[...])
    qe, qo = _unpack_words(q_ref[...])
    ae = jnp.maximum(pe + qe, 0.0).astype(jnp.bfloat16)
    ao = jnp.maximum(po + qo, 0.0).astype(jnp.bfloat16)
    me = (jnp.dot(ae, w2ee[...], preferred_element_type=jnp.float32)
          + jnp.dot(ao, w2oe[...], preferred_element_type=jnp.float32) + b2e[...])
    mo = (jnp.dot(ae, w2eo[...], preferred_element_type=jnp.float32)
          + jnp.dot(ao, w2oo[...], preferred_element_type=jnp.float32) + b2o[...])
    m_ref[...] = _pack_words(me, mo)


def _edge_mlp(P, Q, w2ee, w2eo, w2oe, w2oo, b2e, b2o):
    hd = D // 2
    wspec = lambda: pl.BlockSpec((hd, hd), lambda i: (0, 0))
    return pl.pallas_call(
        _edge_body,
        grid=(NT_E,),
        in_specs=[
            pl.BlockSpec((TE, hd), lambda i: (i, 0)),
            pl.BlockSpec((TE, hd), lambda i: (i, 0)),
            wspec(), wspec(), wspec(), wspec(),
            pl.BlockSpec((1, hd), lambda i: (0, 0)),
            pl.BlockSpec((1, hd), lambda i: (0, 0)),
        ],
        out_specs=pl.BlockSpec((TE, hd), lambda i: (i, 0)),
        out_shape=jax.ShapeDtypeStruct((E, hd), jnp.float32),
    )(P, Q, w2ee, w2eo, w2oe, w2oo, b2e.reshape(1, hd), b2o.reshape(1, hd))


_WR = 320
_SCH = 2000
_CB = 64
_RING = 4096
_FL = 2048
_PLN = E + 2 * _FL
_PAD_PK = _WR


def _sc_mesh_info():
    info = plsc.get_sparse_core_info()
    return info.num_cores, info.num_subcores


def _scatter_plan(dst):
    nc, ns = _sc_mesh_info()
    nw = nc * ns
    nchunks = E // _SCH
    mesh = plsc.VectorSubcoreMesh(core_axis_name="c", subcore_axis_name="s")

    @functools.partial(
        pl.kernel,
        out_type=(jax.ShapeDtypeStruct((nw * _PLN,), jnp.int32),
                  jax.ShapeDtypeStruct((nw * 8,), jnp.int32)),
        mesh=mesh,
        compiler_params=pltpu.CompilerParams(needs_layout_passes=False),
        scratch_types=[
            pltpu.VMEM((_RING,), jnp.int32),
            pltpu.VMEM((_SCH,), jnp.int32),
            pltpu.VMEM((16,), jnp.int32),
        ],
    )
    def k(dst_hbm, plan_hbm, cnt_hbm, ring, dstv, cbuf):
        wid = lax.axis_index("s") * nc + lax.axis_index("c")
        lo = wid * _WR
        base_out = wid * _PLN
        iota = lax.iota(jnp.int32, 16)

        def flush(flushed):
            fl8 = pl.multiple_of(flushed, _FL)
            half = (fl8 >> 11) & 1
            pltpu.sync_copy(ring.at[pl.ds(half * _FL, _FL)],
                            plan_hbm.at[pl.ds(base_out + fl8, _FL)])
            return flushed + _FL

        def chunk(c, carry):
            cntv, flushed = carry
            pltpu.sync_copy(dst_hbm.at[pl.ds(c * _SCH, _SCH)], dstv)

            def scan_g(g, cntv):
                ld = dstv[pl.ds(g * 16, 16)] - lo
                msk = (ld >= 0) & (ld < _WR)
                pk = ((c * _SCH + g * 16 + iota) << 9) | (ld & 511)
                pos = (cntv + plsc.cumsum(msk.astype(jnp.int32)) - 1) & (_RING - 1)
                plsc.store_scatter(ring, [pos], pk, mask=msk)
                return cntv + plsc.all_reduce_population_count(msk)

            cntv = lax.fori_loop(0, _SCH // 16, scan_g, cntv)
            cnt_s = lax.reduce_max(cntv, (0,))

            def maybe_flush(flushed):
                return lax.cond(cnt_s - flushed >= _FL, flush,
                                lambda f: f, flushed)

            return cntv, maybe_flush(flushed)

        cntv, flushed = lax.fori_loop(
            0, nchunks, chunk,
            (jnp.zeros((16,), jnp.int32), jnp.int32(0)))
        cnt_s = lax.reduce_max(cntv, (0,))
        padv = jnp.full((16,), _PAD_PK, jnp.int32)
        for kk in range(4):
            plsc.store_scatter(ring, [(cnt_s + kk * 16 + iota) & (_RING - 1)], padv)

        def cond(fl):
            return fl < cnt_s + _CB

        flushed = lax.while_loop(cond, flush, flushed)
        cbuf[pl.ds(0, 16)] = cntv
        pltpu.sync_copy(cbuf.at[pl.ds(0, 8)], cnt_hbm.at[pl.ds(wid * 8, 8)])

    return k(dst)


def _scatter_max2(Mv, plan, counts):
    nc, ns = _sc_mesh_info()
    nw = nc * ns
    hd = D // 2
    mesh = plsc.VectorSubcoreMesh(core_axis_name="c", subcore_axis_name="s")

    @functools.partial(
        pl.kernel,
        out_type=jax.ShapeDtypeStruct((nw * _WR * D,), jnp.float32),
        mesh=mesh,
        compiler_params=pltpu.CompilerParams(needs_layout_passes=False),
        scratch_types=(
            [pltpu.VMEM(((_WR + 1) * 16,), jnp.float32) for _ in range(16)]
            + [
                pltpu.VMEM((16 * _CB,), jnp.int32),
                pltpu.VMEM((_CB,), jnp.int32),
                pltpu.VMEM((_CB,), jnp.int32),
                pltpu.VMEM((_CB,), jnp.int32),
                pltpu.VMEM((_CB,), jnp.int32),
                pltpu.VMEM((_CB, hd), jnp.float32),
                pltpu.VMEM((_CB, hd), jnp.float32),
                pltpu.VMEM((16,), jnp.int32),
                pltpu.VMEM((40 * D,), jnp.float32),
                pltpu.SemaphoreType.DMA,
                pltpu.SemaphoreType.DMA,
            ]
        ),
    )
    def k(M_hbm, plan_hbm, cnt_hbm, agg_hbm, *rest):
        accs = rest[:16]
        (pkeb, mid0, mid1, mld0, mld1, rows0, rows1, cbuf, stg,
         sem0, sem1) = rest[16:]
        wid = lax.axis_index("s") * nc + lax.axis_index("c")
        lo = wid * _WR
        base_in = wid * _PLN
        iota = lax.iota(jnp.int32, 16)
        neg = jnp.full((16,), _NEG_INF, jnp.float32)
        m16 = jnp.full((16,), -65536, jnp.int32)

        def initacc(i, _):
            for b in range(16):
                plsc.store_scatter(accs[b], [jnp.full((16,), i * 16, jnp.int32) + iota], neg)
            return 0
        lax.fori_loop(0, _WR + 1, initacc, 0)

        cbuf[pl.ds(0, 16)] = jnp.zeros((16,), jnp.int32)
        pltpu.sync_copy(cnt_hbm.at[pl.ds(wid * 8, 8)], cbuf.at[pl.ds(0, 8)])
        cnt = lax.reduce_max(cbuf[pl.ds(0, 16)], (0,))
        nbat = (cnt + _CB - 1) // _CB

        slots = ((mid0, mld0, rows0, sem0),
                 (mid1, mld1, rows1, sem1))

        def fetch_sb(sb):
            off = pl.multiple_of(sb * 16 * _CB, 16 * _CB)
            pltpu.sync_copy(plan_hbm.at[pl.ds(base_in + off, 16 * _CB)], pkeb)

        def stage(b, slot):
            mid, mld, rows, sem = slots[slot]
            lb = pl.multiple_of((b & 15) * _CB, _CB)
            for i in range(_CB // 16):
                pk = pkeb[pl.ds(lb + i * 16, 16)]
                mid[pl.ds(i * 16, 16)] = pk >> 9
                mld[pl.ds(i * 16, 16)] = pk & 511
            return pltpu.async_copy(M_hbm.at[mid], rows, sem)

        def process(slot):
            mld, rows = slots[slot][1], slots[slot][2]

            def group(g, _):
                mldv = jnp.minimum(mld[pl.ds(g * 16, 16)], _WR)

                def rowpair(rp, _):
                    for half in range(2):
                        r2 = rp * 2 + half
                        r = g * 16 + r2
                        rfull = jnp.full((16,), r, jnp.int32)
                        lds = jnp.take(mldv, jnp.full((16,), r2, jnp.int32))
                        addr = lds * 16 + iota
                        for j in range(hd // 16):
                            w = plsc.load_gather(rows, [rfull, iota + j * 16])
                            wi = plsc.bitcast(w, jnp.int32)
                            lov = plsc.bitcast(wi << 16, jnp.float32)
                            hiv = plsc.bitcast(wi & m16, jnp.float32)
                            for par, val in ((0, lov), (1, hiv)):
                                ref = accs[2 * j + par]
                                a = plsc.load_gather(ref, [addr])
                                plsc.store_scatter(ref, [addr], jnp.maximum(a, val))
                    return 0

                lax.fori_loop(0, 8, rowpair, 0)
                return 0

            lax.fori_loop(0, _CB // 16, group, 0)

        @pl.when(nbat > 0)
        def _run():
            fetch_sb(0)
            stage(0, 0)

            def pair(i, _):
                b1 = 2 * i + 1

                @pl.when(b1 < nbat)
                def _s1():
                    stage(b1, 1)

                pltpu.make_async_copy(M_hbm.at[slots[0][0]], slots[0][2],
                                      slots[0][3]).wait()
                process(0)

                @pl.when(b1 + 1 < nbat)
                def _s0():
                    @pl.when(((b1 + 1) & 15) == 0)
                    def _f():
                        fetch_sb((b1 + 1) >> 4)
                    stage(b1 + 1, 0)

                @pl.when(b1 < nbat)
                def _p1():
                    pltpu.make_async_copy(M_hbm.at[slots[1][0]], slots[1][2],
                                          slots[1][3]).wait()
                    process(1)
                return 0

            lax.fori_loop(0, (nbat + 1) // 2, pair, 0)

        def rowchunk(rc, _):
            def rowcp(r2, _):
                rowv = jnp.full((16,), (rc * 40 + r2) * 16, jnp.int32) + iota
                for j in range(8):
                    for par in range(2):
                        v = plsc.load_gather(accs[2 * j + par], [rowv])
                        plsc.store_scatter(
                            stg, [r2 * D + 32 * j + 2 * iota + par], v)
                return 0

            lax.fori_loop(0, 40, rowcp, 0)
            pltpu.sync_copy(
                stg, agg_hbm.at[pl.ds((lo + rc * 40) * D, 40 * D)])
            return 0

        lax.fori_loop(0, _WR // 40, rowchunk, 0)

    return k(Mv, plan, counts)


def _upd_body(h_ref, agg_ref, out_ref):
    agg = agg_ref[...]
    out_ref[...] = h_ref[...] + jnp.where(jnp.isneginf(agg), 0.0, agg)


def _h_update(h, agg):
    return pl.pallas_call(
        _upd_body,
        grid=(NT_N,),
        in_specs=[
            pl.BlockSpec((TN, D), lambda i: (i, 0)),
            pl.BlockSpec((TN, D), lambda i: (i, 0)),
        ],
        out_specs=pl.BlockSpec((TN, D), lambda i: (i, 0)),
        out_shape=jax.ShapeDtypeStruct((N, D), jnp.float32),
    )(h, agg)


def kernel(nodes, coords, edge_index, hn_W1, hn_b1, hn_W2, hn_b2,
           hc_W1, hc_b1, hc_W2, hc_b2, mp_W1, mp_b1, mp_W2, mp_b2):
    src = edge_index[0]
    dst = edge_index[1]
    coords8 = jnp.pad(coords, ((0, 0), (0, 5)))
    hc_W1p = jnp.pad(hc_W1, ((0, 5), (0, 0)))

    h, hcv = _encode(nodes, coords8, hn_W1, hn_b1, hn_W2, hn_b2,
                     hc_W1p, hc_b1, hc_W2, hc_b2)

    plan, counts = _scatter_plan(dst)

    for i in range(3):
        wa = mp_W1[i, 0:D, :]
        wb = mp_W1[i, D:2 * D, :]
        wc = mp_W1[i, 2 * D:3 * D, :]
        w2 = mp_W2[i]
        w2b = w2.astype(jnp.bfloat16)
        S, T = _node_transform(h, hcv,
                               wa[:, 0::2], wa[:, 1::2],
                               wb[:, 0::2], wb[:, 1::2],
                               wc[:, 0::2], wc[:, 1::2],
                               mp_b1[i, 0::2], mp_b1[i, 1::2])
        P, Q = _gather2(S, T, src, dst)
        M = _edge_mlp(P, Q,
                      w2b[0::2, 0::2], w2b[0::2, 1::2],
                      w2b[1::2, 0::2], w2b[1::2, 1::2],
                      mp_b2[i, 0::2], mp_b2[i, 1::2])
        agg = _scatter_max2(M, plan, counts).reshape(-1, D)[:N]
        h = _h_update(h, agg)
    return h
